# baseline (speedup 1.0000x reference)
import functools

import jax
import jax.numpy as jnp
from jax import lax
from jax.experimental import pallas as pl
from jax.experimental.pallas import tpu as pltpu
from jax.experimental.pallas import tpu_sc as plsc

NC = 2
NS = 16
NW = NC * NS
CH = 128
N = 10000
NPAD = 10240
HR = NPAD // 128
E = 320000

_mesh = plsc.VectorSubcoreMesh(core_axis_name="c", subcore_axis_name="s")
_f32 = jnp.float32


def _zero16():
    return jnp.zeros((16,), _f32)


def _ones16():
    return jnp.ones((16,), _f32)


def _deg_body(ew, dst_hbm, out_hbm, dstv, hist):
    cid = lax.axis_index("c")
    sid = lax.axis_index("s")
    wid = sid * NC + cid

    def _zrow(i, c):
        hist[pl.ds(pl.multiple_of(i * 16, 8), 16)] = _zero16()
        return c

    lax.fori_loop(0, NPAD // 16, _zrow, 0)

    def _chunk(c, carry):
        off = pl.multiple_of(wid * ew + c * CH, 8)
        pltpu.sync_copy(dst_hbm.at[pl.ds(off, CH)], dstv)
        for j in range(CH // 16):
            dv = dstv[pl.ds(j * 16, 16)]
            plsc.addupdate_scatter(hist, [dv], _ones16())
        return carry

    lax.fori_loop(0, ew // CH, _chunk, 0)

    pltpu.sync_copy(hist,
                    out_hbm.at[pl.ds(pl.multiple_of(wid * NPAD, 8), NPAD)])


def _agg_body(nch, g_hbm, src_hbm, dst_hbm, out_hbm, sb0, sb1, dstv,
              rows0, rows1, out_sh, gs0, gs1, ss0, ss1):
    cid = lax.axis_index("c")
    sid = lax.axis_index("s")
    wid = sid * NC + cid
    rows_per_tile = NPAD // NS

    pltpu.sync_copy(dst_hbm.at[wid], dstv)

    def _zrow(i, c):
        for j in range(8):
            rows0[i, pl.ds(j * 16, 16)] = _zero16()
        return c

    lax.fori_loop(0, CH, _zrow, 0)
    for r in range(rows_per_tile // CH):
        pltpu.sync_copy(rows0, out_sh.at[pl.ds(
            pl.multiple_of(sid * rows_per_tile + r * CH, 8), CH)])
    plsc.subcore_barrier()

    sbs = (sb0, sb1)
    rbs = (rows0, rows1)
    gss = (gs0, gs1)
    sss = (ss0, ss1)

    pltpu.sync_copy(src_hbm.at[wid, 0], sb0)
    pltpu.async_copy(g_hbm.at[sb0], rows0, gs0)
    pltpu.async_copy(src_hbm.at[wid, 1], sb1, ss1)

    def _pair(i, carry):
        for k in range(2):
            c = i * 2 + k
            ko = (k + 1) % 2
            pltpu.make_async_copy(src_hbm.at[wid, c], sbs[ko],
                                  sss[ko]).wait()
            pltpu.async_copy(g_hbm.at[sbs[ko]], rbs[ko], gss[ko])
            pltpu.make_async_copy(g_hbm.at[sbs[k]], rbs[k], gss[k]).wait()
            pltpu.sync_copy(rbs[k], out_sh.at[dstv.at[c]], add=True)
            nxt = jnp.minimum(c + 2, nch - 1)
            pltpu.async_copy(src_hbm.at[wid, nxt], sbs[k], sss[k])
        return carry

    lax.fori_loop(0, nch // 2, _pair, 0)
    pltpu.make_async_copy(g_hbm.at[sbs[0]], rbs[0], gss[0]).wait()
    pltpu.make_async_copy(src_hbm.at[wid, 0], sbs[1], sss[1]).wait()
    plsc.subcore_barrier()

    for r in range(rows_per_tile // CH):
        sl = pl.ds(pl.multiple_of(sid * rows_per_tile + r * CH, 8), CH)
        pltpu.sync_copy(out_sh.at[sl], rows0)
        pltpu.sync_copy(rows0, out_hbm.at[pl.ds(
            pl.multiple_of(cid * NPAD + sid * rows_per_tile + r * CH, 8),
            CH)])


def _make_deg(ew):
    return pl.kernel(
        functools.partial(_deg_body, ew),
        out_type=jax.ShapeDtypeStruct((NW * NPAD,), _f32),
        mesh=_mesh,
        scratch_types=[
            pltpu.VMEM((CH,), jnp.int32),
            pltpu.VMEM((NPAD,), _f32),
        ],
        compiler_params=pltpu.CompilerParams(needs_layout_passes=False),
    )


def _make_agg(nch):
    return pl.kernel(
        functools.partial(_agg_body, nch),
        out_type=jax.ShapeDtypeStruct((NC * NPAD, 128), _f32),
        mesh=_mesh,
        scratch_types=[
            pltpu.VMEM((CH,), jnp.int32),
            pltpu.VMEM((CH,), jnp.int32),
            pltpu.VMEM((nch, CH), jnp.int32),
            pltpu.VMEM((CH, 128), _f32),
            pltpu.VMEM((CH, 128), _f32),
            pltpu.VMEM_SHARED((NPAD, 128), _f32),
            pltpu.SemaphoreType.DMA,
            pltpu.SemaphoreType.DMA,
            pltpu.SemaphoreType.DMA,
            pltpu.SemaphoreType.DMA,
        ],
        compiler_params=pltpu.CompilerParams(needs_layout_passes=False),
    )


def _tc1_body(x_ref, w_ref, d_ref, g_ref, dis_ref):
    deg = jnp.sum(d_ref[...], axis=0) + 1.0
    disc = lax.rsqrt(deg)
    dis_ref[...] = disc
    h = jnp.dot(x_ref[...], w_ref[...], preferred_element_type=_f32)
    g_ref[...] = h * disc


def _tc_mid_body(e0_ref, e1_ref, g_ref, dis_ref, b_ref, w_ref, go_ref):
    disc = dis_ref[...]
    t = (e0_ref[...] + e1_ref[...] + g_ref[...]) * disc + b_ref[...]
    xn = jnp.maximum(t, 0.0)
    h = jnp.dot(xn, w_ref[...], preferred_element_type=_f32)
    go_ref[...] = h * disc


def _tc_fin_body(e0_ref, e1_ref, g_ref, dis_ref, b_ref, w_ref, b4_ref,
                 o_ref):
    disc = dis_ref[...]
    t = (e0_ref[...] + e1_ref[...] + g_ref[...]) * disc + b_ref[...]
    xn = jnp.maximum(t, 0.0)
    o_ref[...] = jnp.dot(xn, w_ref[...],
                         preferred_element_type=_f32) + b4_ref[...]


_GRID = NPAD // 128

_blk_rows = pl.BlockSpec((128, 128), lambda b: (b, 0))
_blk_w = pl.BlockSpec((128, 128), lambda b: (0, 0))
_blk_dis = pl.BlockSpec((128, 1), lambda b: (b, 0))
_blk_bias = pl.BlockSpec((1, 128), lambda b: (0, 0))


def _tc1(x_pad, W1, deg_all):
    return pl.pallas_call(
        _tc1_body,
        grid=(_GRID,),
        in_specs=[
            _blk_rows,
            _blk_w,
            pl.BlockSpec((NW, 128, 1), lambda b: (0, b, 0)),
        ],
        out_specs=[_blk_rows, _blk_dis],
        out_shape=[
            jax.ShapeDtypeStruct((NPAD, 128), _f32),
            jax.ShapeDtypeStruct((NPAD, 1), _f32),
        ],
    )(x_pad, W1, deg_all)


def _tc_mid(e_flat, g_prev, disp, b_row, W):
    return pl.pallas_call(
        _tc_mid_body,
        grid=(_GRID,),
        in_specs=[
            pl.BlockSpec((128, 128), lambda b: (b, 0)),
            pl.BlockSpec((128, 128), lambda b: (b + _GRID, 0)),
            _blk_rows,
            _blk_dis,
            _blk_bias,
            _blk_w,
        ],
        out_specs=_blk_rows,
        out_shape=jax.ShapeDtypeStruct((NPAD, 128), _f32),
    )(e_flat, e_flat, g_prev, disp, b_row, W)


def _tc_fin(e_flat, g_prev, disp, b_row, W4, b4_row):
    dout = W4.shape[1]
    return pl.pallas_call(
        _tc_fin_body,
        grid=(_GRID,),
        in_specs=[
            pl.BlockSpec((128, 128), lambda b: (b, 0)),
            pl.BlockSpec((128, 128), lambda b: (b + _GRID, 0)),
            _blk_rows,
            _blk_dis,
            _blk_bias,
            pl.BlockSpec((128, dout), lambda b: (0, 0)),
            pl.BlockSpec((1, dout), lambda b: (0, 0)),
        ],
        out_specs=pl.BlockSpec((128, dout), lambda b: (b, 0)),
        out_shape=jax.ShapeDtypeStruct((NPAD, dout), _f32),
    )(e_flat, e_flat, g_prev, disp, b_row, W4, b4_row)


def kernel(x, edge_index, W1, b1, W2, b2, W3, b3, W4, b4):
    src = edge_index[0]
    dst = edge_index[1]

    nch = ((E + NW * CH - 1) // (NW * CH) + 1) // 2 * 2
    e_pad = NW * nch * CH
    padn = e_pad - E
    ew = nch * CH
    pad_src = jnp.zeros((padn,), jnp.int32)
    pad_dst = (N + (jnp.arange(padn, dtype=jnp.int32) % (NPAD - N)))
    src_pad = jnp.concatenate([src, pad_src]).reshape(NW, nch, CH)
    dst_pad = jnp.concatenate([dst, pad_dst]).reshape(NW, nch, CH)
    dst_flat = jnp.concatenate([dst, pad_dst])

    x_pad = jnp.concatenate(
        [x, jnp.zeros((NPAD - N, x.shape[1]), _f32)])

    deg_all = _make_deg(ew)(dst_flat).reshape(NW, NPAD, 1)
    g1, disp = _tc1(x_pad, W1, deg_all)

    b1r = b1.reshape(1, -1)
    b2r = b2.reshape(1, -1)
    b3r = b3.reshape(1, -1)
    b4r = b4.reshape(1, -1)

    agg = _make_agg(nch)
    e1f = agg(g1, src_pad, dst_pad)
    g2 = _tc_mid(e1f, g1, disp, b1r, W2)
    e2f = agg(g2, src_pad, dst_pad)
    g3 = _tc_mid(e2f, g2, disp, b2r, W3)
    e3f = agg(g3, src_pad, dst_pad)
    out = _tc_fin(e3f, g3, disp, b3r, W4, b4r)
    return out[:N]


# speedup vs baseline: 3.6198x; 3.6198x over previous
import functools

import jax
import jax.numpy as jnp
from jax import lax
from jax.experimental import pallas as pl
from jax.experimental.pallas import tpu as pltpu
from jax.experimental.pallas import tpu_sc as plsc

NC = 2
NS = 16
NW = NC * NS
CH = 128
N = 10000
NPAD = 10240
HR = NPAD // 128
E = 320000

_mesh = plsc.VectorSubcoreMesh(core_axis_name="c", subcore_axis_name="s")
_f32 = jnp.float32


def _zero16():
    return jnp.zeros((16,), _f32)


def _ones16():
    return jnp.ones((16,), _f32)


def _deg_body(ew, dst_hbm, out_hbm, dstv, hist):
    cid = lax.axis_index("c")
    sid = lax.axis_index("s")
    wid = sid * NC + cid

    def _zrow(i, c):
        hist[pl.ds(pl.multiple_of(i * 16, 8), 16)] = _zero16()
        return c

    lax.fori_loop(0, NPAD // 16, _zrow, 0)

    def _chunk(c, carry):
        off = pl.multiple_of(wid * ew + c * CH, 8)
        pltpu.sync_copy(dst_hbm.at[pl.ds(off, CH)], dstv)
        for j in range(CH // 16):
            dv = dstv[pl.ds(j * 16, 16)]
            plsc.addupdate_scatter(hist, [dv], _ones16())
        return carry

    lax.fori_loop(0, ew // CH, _chunk, 0)

    pltpu.sync_copy(hist,
                    out_hbm.at[pl.ds(pl.multiple_of(wid * NPAD, 8), NPAD)])


def _make_deg(ew):
    return pl.kernel(
        functools.partial(_deg_body, ew),
        out_type=jax.ShapeDtypeStruct((NW * NPAD,), _f32),
        mesh=_mesh,
        scratch_types=[
            pltpu.VMEM((CH,), jnp.int32),
            pltpu.VMEM((NPAD,), _f32),
        ],
        compiler_params=pltpu.CompilerParams(needs_layout_passes=False),
    )


def _agg_body(nch, g_hbm, src_hbm, dst_hbm, out_hbm, sb0, sb1, dstv,
              rows0, rows1, out_sh, gs0, gs1, ss0, ss1):
    cid = lax.axis_index("c")
    sid = lax.axis_index("s")
    wid = sid * NC + cid
    rpt = NPAD // NS

    pltpu.sync_copy(dst_hbm.at[wid], dstv)

    def _zrow(i, c):
        for j in range(8):
            rows0[i, pl.ds(j * 16, 16)] = _zero16()
        return c

    lax.fori_loop(0, CH, _zrow, 0)
    for r in range(rpt // CH):
        pltpu.sync_copy(rows0, out_sh.at[pl.ds(
            pl.multiple_of(sid * rpt + r * CH, 8), CH)])
    plsc.subcore_barrier()

    sbs = (sb0, sb1)
    rbs = (rows0, rows1)
    gss = (gs0, gs1)
    sss = (ss0, ss1)

    pltpu.sync_copy(src_hbm.at[wid, 0], sb0)
    pltpu.async_copy(g_hbm.at[sb0], rows0, gs0)
    pltpu.async_copy(src_hbm.at[wid, 1], sb1, ss1)

    def _pair(i, carry):
        for k in range(2):
            c = i * 2 + k
            ko = (k + 1) % 2
            pltpu.make_async_copy(src_hbm.at[wid, c], sbs[ko],
                                  sss[ko]).wait()
            pltpu.async_copy(g_hbm.at[sbs[ko]], rbs[ko], gss[ko])
            pltpu.make_async_copy(g_hbm.at[sbs[k]], rbs[k], gss[k]).wait()
            pltpu.sync_copy(rbs[k], out_sh.at[dstv.at[c]], add=True)
            nxt = jnp.minimum(c + 2, nch - 1)
            pltpu.async_copy(src_hbm.at[wid, nxt], sbs[k], sss[k])
        return carry

    lax.fori_loop(0, nch // 2, _pair, 0)
    pltpu.make_async_copy(g_hbm.at[sbs[0]], rbs[0], gss[0]).wait()
    pltpu.make_async_copy(src_hbm.at[wid, 0], sbs[1], sss[1]).wait()
    plsc.subcore_barrier()

    for r in range(rpt // CH):
        sl = pl.ds(pl.multiple_of(sid * rpt + r * CH, 8), CH)
        pltpu.sync_copy(out_sh.at[sl], rows0)
        pltpu.sync_copy(rows0, out_hbm.at[pl.ds(
            pl.multiple_of(cid * NPAD + sid * rpt + r * CH, 8), CH)])


def _make_agg(nch):
    return pl.kernel(
        functools.partial(_agg_body, nch),
        out_type=jax.ShapeDtypeStruct((NC * NPAD, 128), _f32),
        mesh=_mesh,
        scratch_types=[
            pltpu.VMEM((CH,), jnp.int32),
            pltpu.VMEM((CH,), jnp.int32),
            pltpu.VMEM((nch, CH), jnp.int32),
            pltpu.VMEM((CH, 128), _f32),
            pltpu.VMEM((CH, 128), _f32),
            pltpu.VMEM_SHARED((NPAD, 128), _f32),
            pltpu.SemaphoreType.DMA,
            pltpu.SemaphoreType.DMA,
            pltpu.SemaphoreType.DMA,
            pltpu.SemaphoreType.DMA,
        ],
        compiler_params=pltpu.CompilerParams(needs_layout_passes=False),
    )


BR = 512
GB = NPAD // BR


def _tca_body(d_ref, o_ref):
    degs = jnp.sum(d_ref[...], axis=0) + 1.0
    dis = lax.rsqrt(degs)
    ones = jnp.ones((1, 128), _f32)
    parts = []
    for r in range(8):
        row = dis[r:r + 1, :]
        parts.append(lax.dot_general(
            row, ones, (((0,), (0,)), ((), ())),
            preferred_element_type=_f32))
    o_ref[...] = jnp.concatenate(parts, axis=0)


def _tc1_body(x_ref, w_ref, db_ref, g_ref):
    h = jnp.dot(x_ref[...], w_ref[...], preferred_element_type=_f32)
    g_ref[...] = h * db_ref[...]


def _tc_mid_body(e0_ref, e1_ref, g_ref, db_ref, b_ref, w_ref, go_ref):
    t = ((e0_ref[...] + e1_ref[...] + g_ref[...]) * db_ref[...]
         + b_ref[...])
    xn = jnp.maximum(t, 0.0)
    h = jnp.dot(xn, w_ref[...], preferred_element_type=_f32)
    go_ref[...] = h * db_ref[...]


def _tc_fin_body(e0_ref, e1_ref, g_ref, db_ref, b_ref, w_ref, b4_ref,
                 o_ref):
    t = ((e0_ref[...] + e1_ref[...] + g_ref[...]) * db_ref[...]
         + b_ref[...])
    xn = jnp.maximum(t, 0.0)
    o_ref[...] = jnp.dot(xn, w_ref[...],
                         preferred_element_type=_f32) + b4_ref[...]


_blk_x = pl.BlockSpec((BR, 128), lambda b: (b, 0))
_blk_w = pl.BlockSpec((128, 128), lambda b: (0, 0))
_blk_elo = pl.BlockSpec((BR, 128), lambda b: (b, 0))
_blk_ehi = pl.BlockSpec((BR, 128), lambda b: (b + GB, 0))
_blk_bias = pl.BlockSpec((1, 128), lambda b: (0, 0))


def _tca(deg3):
    return pl.pallas_call(
        _tca_body,
        grid=(HR // 8,),
        in_specs=[pl.BlockSpec((NW, 8, 128), lambda b: (0, b, 0))],
        out_specs=pl.BlockSpec((1024, 128), lambda b: (b, 0)),
        out_shape=jax.ShapeDtypeStruct((NPAD, 128), _f32),
    )(deg3)


def _tc1(x_pad, W1, disb):
    return pl.pallas_call(
        _tc1_body,
        grid=(GB,),
        in_specs=[_blk_x, _blk_w, _blk_x],
        out_specs=_blk_x,
        out_shape=jax.ShapeDtypeStruct((NPAD, 128), _f32),
    )(x_pad, W1, disb)


def _tc_mid(e2, g_prev, disb, b_row, W):
    return pl.pallas_call(
        _tc_mid_body,
        grid=(GB,),
        in_specs=[_blk_elo, _blk_ehi, _blk_x, _blk_x, _blk_bias, _blk_w],
        out_specs=_blk_x,
        out_shape=jax.ShapeDtypeStruct((NPAD, 128), _f32),
    )(e2, e2, g_prev, disb, b_row, W)


def _tc_fin(e2, g_prev, disb, b_row, W4, b4_row):
    dout = W4.shape[1]
    return pl.pallas_call(
        _tc_fin_body,
        grid=(GB,),
        in_specs=[_blk_elo, _blk_ehi, _blk_x, _blk_x, _blk_bias,
                  pl.BlockSpec((128, dout), lambda b: (0, 0)),
                  pl.BlockSpec((1, dout), lambda b: (0, 0))],
        out_specs=pl.BlockSpec((BR, dout), lambda b: (b, 0)),
        out_shape=jax.ShapeDtypeStruct((NPAD, dout), _f32),
    )(e2, e2, g_prev, disb, b_row, W4, b4_row)


def kernel(x, edge_index, W1, b1, W2, b2, W3, b3, W4, b4):
    src = edge_index[0]
    dst = edge_index[1]

    nch = ((E + NW * CH - 1) // (NW * CH) + 7) // 8 * 8
    e_pad = NW * nch * CH
    padn = e_pad - E
    ew = nch * CH
    pidx = jnp.arange(padn, dtype=jnp.int32) % (NPAD - N)
    pad_src = N + pidx
    pad_dst = N + pidx
    src_flat = jnp.concatenate([src, pad_src])
    dst_flat = jnp.concatenate([dst, pad_dst])
    src3 = src_flat.reshape(NW, nch, CH)
    dst3 = dst_flat.reshape(NW, nch, CH)

    x_pad = jnp.concatenate(
        [x, jnp.zeros((NPAD - N, x.shape[1]), _f32)])

    deg3 = _make_deg(ew)(dst_flat).reshape(NW, HR, 128)
    disb = _tca(deg3)
    g1 = _tc1(x_pad, W1, disb)

    b1r = b1.reshape(1, -1)
    b2r = b2.reshape(1, -1)
    b3r = b3.reshape(1, -1)
    b4r = b4.reshape(1, -1)

    agg = _make_agg(nch)
    e1 = agg(g1, src3, dst3)
    g2 = _tc_mid(e1, g1, disb, b1r, W2)
    e2 = agg(g2, src3, dst3)
    g3 = _tc_mid(e2, g2, disb, b2r, W3)
    e3 = agg(g3, src3, dst3)
    out = _tc_fin(e3, g3, disb, b3r, W4, b4r)
    return out[:N]


# speedup vs baseline: 3.7054x; 1.0236x over previous
import functools

import jax
import jax.numpy as jnp
from jax import lax
from jax.experimental import pallas as pl
from jax.experimental.pallas import tpu as pltpu
from jax.experimental.pallas import tpu_sc as plsc

NC = 2
NS = 16
NW = NC * NS
CH = 128
N = 10000
NPAD = 10240
HR = NPAD // 128
E = 320000

_mesh = plsc.VectorSubcoreMesh(core_axis_name="c", subcore_axis_name="s")
_f32 = jnp.float32


def _zero16():
    return jnp.zeros((16,), _f32)


def _ones16():
    return jnp.ones((16,), _f32)


def _deg_body(ew, dst_hbm, out_hbm, dstv, hist):
    cid = lax.axis_index("c")
    sid = lax.axis_index("s")
    wid = sid * NC + cid

    def _zrow(i, c):
        hist[pl.ds(pl.multiple_of(i * 16, 8), 16)] = _zero16()
        return c

    lax.fori_loop(0, NPAD // 16, _zrow, 0)

    def _chunk(c, carry):
        off = pl.multiple_of(wid * ew + c * CH, 8)
        pltpu.sync_copy(dst_hbm.at[pl.ds(off, CH)], dstv)
        for j in range(CH // 16):
            dv = dstv[pl.ds(j * 16, 16)]
            plsc.addupdate_scatter(hist, [dv], _ones16())
        return carry

    lax.fori_loop(0, ew // CH, _chunk, 0)

    pltpu.sync_copy(hist,
                    out_hbm.at[pl.ds(pl.multiple_of(wid * NPAD, 8), NPAD)])


def _make_deg(ew):
    return pl.kernel(
        functools.partial(_deg_body, ew),
        out_type=jax.ShapeDtypeStruct((NW * NPAD,), _f32),
        mesh=_mesh,
        scratch_types=[
            pltpu.VMEM((CH,), jnp.int32),
            pltpu.VMEM((NPAD,), _f32),
        ],
        compiler_params=pltpu.CompilerParams(needs_layout_passes=False),
    )


def _agg_body(nch, g_hbm, src_hbm, dst_hbm, out_hbm, sb0, sb1, dstv,
              rows0, rows1, out_sh, gs0, gs1, ss0, ss1):
    cid = lax.axis_index("c")
    sid = lax.axis_index("s")
    wid = sid * NC + cid
    rpt = NPAD // NS

    pltpu.sync_copy(dst_hbm.at[wid], dstv)

    def _zrow(i, c):
        for j in range(8):
            rows0[i, pl.ds(j * 16, 16)] = _zero16()
        return c

    lax.fori_loop(0, CH, _zrow, 0)
    for r in range(rpt // CH):
        pltpu.sync_copy(rows0, out_sh.at[pl.ds(
            pl.multiple_of(sid * rpt + r * CH, 8), CH)])
    plsc.subcore_barrier()

    sbs = (sb0, sb1)
    rbs = (rows0, rows1)
    gss = (gs0, gs1)
    sss = (ss0, ss1)

    pltpu.sync_copy(src_hbm.at[wid, 0], sb0)
    pltpu.async_copy(g_hbm.at[sb0], rows0, gs0)
    pltpu.async_copy(src_hbm.at[wid, 1], sb1, ss1)

    def _pair(i, carry):
        for k in range(2):
            c = i * 2 + k
            ko = (k + 1) % 2
            pltpu.make_async_copy(src_hbm.at[wid, c], sbs[ko],
                                  sss[ko]).wait()
            pltpu.async_copy(g_hbm.at[sbs[ko]], rbs[ko], gss[ko])
            pltpu.make_async_copy(g_hbm.at[sbs[k]], rbs[k], gss[k]).wait()
            pltpu.sync_copy(rbs[k], out_sh.at[dstv.at[c]], add=True)
            nxt = jnp.minimum(c + 2, nch - 1)
            pltpu.async_copy(src_hbm.at[wid, nxt], sbs[k], sss[k])
        return carry

    lax.fori_loop(0, nch // 2, _pair, 0)
    pltpu.make_async_copy(g_hbm.at[sbs[0]], rbs[0], gss[0]).wait()
    pltpu.make_async_copy(src_hbm.at[wid, 0], sbs[1], sss[1]).wait()
    plsc.subcore_barrier()

    for r in range(rpt // CH):
        sl = pl.ds(pl.multiple_of(sid * rpt + r * CH, 8), CH)
        pltpu.sync_copy(out_sh.at[sl], rows0)
        pltpu.sync_copy(rows0, out_hbm.at[pl.ds(
            pl.multiple_of(cid * NPAD + sid * rpt + r * CH, 8), CH)])


def _make_agg(nch):
    return pl.kernel(
        functools.partial(_agg_body, nch),
        out_type=jax.ShapeDtypeStruct((NC * NPAD, 128), _f32),
        mesh=_mesh,
        scratch_types=[
            pltpu.VMEM((CH,), jnp.int32),
            pltpu.VMEM((CH,), jnp.int32),
            pltpu.VMEM((nch, CH), jnp.int32),
            pltpu.VMEM((CH, 128), _f32),
            pltpu.VMEM((CH, 128), _f32),
            pltpu.VMEM_SHARED((NPAD, 128), _f32),
            pltpu.SemaphoreType.DMA,
            pltpu.SemaphoreType.DMA,
            pltpu.SemaphoreType.DMA,
            pltpu.SemaphoreType.DMA,
        ],
        compiler_params=pltpu.CompilerParams(needs_layout_passes=False),
    )


BR = 512
GB = NPAD // BR


def _tca_body(d_ref, h_ref, db_ref, g_ref):
    degs = jnp.sum(d_ref[...], axis=0) + 1.0
    dis = lax.rsqrt(degs)
    ones = jnp.ones((1, 128), _f32)
    parts = []
    for r in range(8):
        row = dis[r:r + 1, :]
        parts.append(lax.dot_general(
            row, ones, (((0,), (0,)), ((), ())),
            preferred_element_type=_f32))
    db = jnp.concatenate(parts, axis=0)
    db_ref[...] = db
    g_ref[...] = h_ref[...] * db


def _tch_body(x_ref, w_ref, h_ref):
    h_ref[...] = jnp.dot(x_ref[...], w_ref[...],
                         preferred_element_type=_f32)


def _tc_mid_body(e0_ref, e1_ref, g_ref, db_ref, b_ref, w_ref, go_ref):
    t = ((e0_ref[...] + e1_ref[...] + g_ref[...]) * db_ref[...]
         + b_ref[...])
    xn = jnp.maximum(t, 0.0)
    h = jnp.dot(xn, w_ref[...], preferred_element_type=_f32)
    go_ref[...] = h * db_ref[...]


def _tc_fin_body(e0_ref, e1_ref, g_ref, db_ref, b_ref, w_ref, b4_ref,
                 o_ref):
    t = ((e0_ref[...] + e1_ref[...] + g_ref[...]) * db_ref[...]
         + b_ref[...])
    xn = jnp.maximum(t, 0.0)
    o_ref[...] = jnp.dot(xn, w_ref[...],
                         preferred_element_type=_f32) + b4_ref[...]


_blk_x = pl.BlockSpec((BR, 128), lambda b: (b, 0))
_blk_w = pl.BlockSpec((128, 128), lambda b: (0, 0))
_blk_elo = pl.BlockSpec((BR, 128), lambda b: (b, 0))
_blk_ehi = pl.BlockSpec((BR, 128), lambda b: (b + GB, 0))
_blk_bias = pl.BlockSpec((1, 128), lambda b: (0, 0))


def _tca(deg3, h1):
    return pl.pallas_call(
        _tca_body,
        grid=(HR // 8,),
        in_specs=[pl.BlockSpec((NW, 8, 128), lambda b: (0, b, 0)),
                  pl.BlockSpec((1024, 128), lambda b: (b, 0))],
        out_specs=[pl.BlockSpec((1024, 128), lambda b: (b, 0)),
                   pl.BlockSpec((1024, 128), lambda b: (b, 0))],
        out_shape=[
            jax.ShapeDtypeStruct((NPAD, 128), _f32),
            jax.ShapeDtypeStruct((NPAD, 128), _f32),
        ],
    )(deg3, h1)


def _tch(x_pad, W1):
    return pl.pallas_call(
        _tch_body,
        grid=(GB,),
        in_specs=[_blk_x, _blk_w],
        out_specs=_blk_x,
        out_shape=jax.ShapeDtypeStruct((NPAD, 128), _f32),
    )(x_pad, W1)


def _tc_mid(e2, g_prev, disb, b_row, W):
    return pl.pallas_call(
        _tc_mid_body,
        grid=(GB,),
        in_specs=[_blk_elo, _blk_ehi, _blk_x, _blk_x, _blk_bias, _blk_w],
        out_specs=_blk_x,
        out_shape=jax.ShapeDtypeStruct((NPAD, 128), _f32),
    )(e2, e2, g_prev, disb, b_row, W)


def _tc_fin(e2, g_prev, disb, b_row, W4, b4_row):
    dout = W4.shape[1]
    return pl.pallas_call(
        _tc_fin_body,
        grid=(GB,),
        in_specs=[_blk_elo, _blk_ehi, _blk_x, _blk_x, _blk_bias,
                  pl.BlockSpec((128, dout), lambda b: (0, 0)),
                  pl.BlockSpec((1, dout), lambda b: (0, 0))],
        out_specs=pl.BlockSpec((BR, dout), lambda b: (b, 0)),
        out_shape=jax.ShapeDtypeStruct((NPAD, dout), _f32),
    )(e2, e2, g_prev, disb, b_row, W4, b4_row)


def kernel(x, edge_index, W1, b1, W2, b2, W3, b3, W4, b4):
    src = edge_index[0]
    dst = edge_index[1]

    nch = ((E + NW * CH - 1) // (NW * CH) + 7) // 8 * 8
    e_pad = NW * nch * CH
    padn = e_pad - E
    ew = nch * CH
    pidx = jnp.arange(padn, dtype=jnp.int32) % (NPAD - N)
    pad_src = N + pidx
    pad_dst = N + pidx
    src_flat = jnp.concatenate([src, pad_src])
    dst_flat = jnp.concatenate([dst, pad_dst])
    src3 = src_flat.reshape(NW, nch, CH)
    dst3 = dst_flat.reshape(NW, nch, CH)

    x_pad = jnp.concatenate(
        [x, jnp.zeros((NPAD - N, x.shape[1]), _f32)])

    h1 = _tch(x_pad, W1)
    deg3 = _make_deg(ew)(dst_flat).reshape(NW, HR, 128)
    disb, g1 = _tca(deg3, h1)

    b1r = b1.reshape(1, -1)
    b2r = b2.reshape(1, -1)
    b3r = b3.reshape(1, -1)
    b4r = b4.reshape(1, -1)

    agg = _make_agg(nch)
    e1 = agg(g1, src3, dst3)
    g2 = _tc_mid(e1, g1, disb, b1r, W2)
    e2 = agg(g2, src3, dst3)
    g3 = _tc_mid(e2, g2, disb, b2r, W3)
    e3 = agg(g3, src3, dst3)
    out = _tc_fin(e3, g3, disb, b3r, W4, b4r)
    return out[:N]


# speedup vs baseline: 3.8511x; 1.0393x over previous
import functools

import jax
import jax.numpy as jnp
from jax import lax
from jax.experimental import pallas as pl
from jax.experimental.pallas import tpu as pltpu
from jax.experimental.pallas import tpu_sc as plsc

NC = 2
NS = 16
NW = NC * NS
CH = 128
N = 10000
NPAD = 10240
HR = NPAD // 128
E = 320000

_mesh = plsc.VectorSubcoreMesh(core_axis_name="c", subcore_axis_name="s")
_f32 = jnp.float32


def _zero16():
    return jnp.zeros((16,), _f32)


def _ones16():
    return jnp.ones((16,), _f32)


def _deg_body(nch, dst_hbm, out_hbm, db0, db1, hist, sd0, sd1):
    cid = lax.axis_index("c")
    sid = lax.axis_index("s")
    wid = sid * NC + cid

    def _zrow(i, c):
        hist[pl.ds(pl.multiple_of(i * 16, 8), 16)] = _zero16()
        return c

    lax.fori_loop(0, NPAD // 16, _zrow, 0)

    dbs = (db0, db1)
    sds = (sd0, sd1)
    pltpu.async_copy(dst_hbm.at[wid, 0], db0, sd0)
    pltpu.async_copy(dst_hbm.at[wid, 1], db1, sd1)

    def _pair(i, carry):
        for k in range(2):
            c = i * 2 + k
            pltpu.make_async_copy(dst_hbm.at[wid, c], dbs[k],
                                  sds[k]).wait()
            for j in range(CH // 16):
                dv = dbs[k][pl.ds(j * 16, 16)]
                plsc.addupdate_scatter(hist, [dv], _ones16())
            nxt = jnp.minimum(c + 2, nch - 1)
            pltpu.async_copy(dst_hbm.at[wid, nxt], dbs[k], sds[k])
        return carry

    lax.fori_loop(0, nch // 2, _pair, 0)
    pltpu.make_async_copy(dst_hbm.at[wid, 0], db0, sd0).wait()
    pltpu.make_async_copy(dst_hbm.at[wid, 0], db1, sd1).wait()

    pltpu.sync_copy(hist,
                    out_hbm.at[pl.ds(pl.multiple_of(wid * NPAD, 8), NPAD)])


def _make_deg(nch):
    return pl.kernel(
        functools.partial(_deg_body, nch),
        out_type=jax.ShapeDtypeStruct((NW * NPAD,), _f32),
        mesh=_mesh,
        scratch_types=[
            pltpu.VMEM((CH,), jnp.int32),
            pltpu.VMEM((CH,), jnp.int32),
            pltpu.VMEM((NPAD,), _f32),
            pltpu.SemaphoreType.DMA,
            pltpu.SemaphoreType.DMA,
        ],
        compiler_params=pltpu.CompilerParams(needs_layout_passes=False),
    )


def _agg_body(nch, g_hbm, src_hbm, dst_hbm, out_hbm, sb0, sb1, dstv,
              rows0, rows1, out_sh, gs0, gs1, ss0, ss1):
    cid = lax.axis_index("c")
    sid = lax.axis_index("s")
    wid = sid * NC + cid
    rpt = NPAD // NS

    pltpu.sync_copy(dst_hbm.at[wid], dstv)

    def _zrow(i, c):
        for j in range(8):
            rows0[i, pl.ds(j * 16, 16)] = _zero16()
        return c

    lax.fori_loop(0, CH, _zrow, 0)
    for r in range(rpt // CH):
        pltpu.sync_copy(rows0, out_sh.at[pl.ds(
            pl.multiple_of(sid * rpt + r * CH, 8), CH)])
    plsc.subcore_barrier()

    sbs = (sb0, sb1)
    rbs = (rows0, rows1)
    gss = (gs0, gs1)
    sss = (ss0, ss1)

    pltpu.sync_copy(src_hbm.at[wid, 0], sb0)
    pltpu.async_copy(g_hbm.at[sb0], rows0, gs0)
    pltpu.async_copy(src_hbm.at[wid, 1], sb1, ss1)

    def _pair(i, carry):
        for k in range(2):
            c = i * 2 + k
            ko = (k + 1) % 2
            pltpu.make_async_copy(src_hbm.at[wid, c], sbs[ko],
                                  sss[ko]).wait()
            pltpu.async_copy(g_hbm.at[sbs[ko]], rbs[ko], gss[ko])
            pltpu.make_async_copy(g_hbm.at[sbs[k]], rbs[k], gss[k]).wait()
            pltpu.sync_copy(rbs[k], out_sh.at[dstv.at[c]], add=True)
            nxt = jnp.minimum(c + 2, nch - 1)
            pltpu.async_copy(src_hbm.at[wid, nxt], sbs[k], sss[k])
        return carry

    lax.fori_loop(0, nch // 2, _pair, 0)
    pltpu.make_async_copy(g_hbm.at[sbs[0]], rbs[0], gss[0]).wait()
    pltpu.make_async_copy(src_hbm.at[wid, 0], sbs[1], sss[1]).wait()
    plsc.subcore_barrier()

    for r in range(rpt // CH):
        sl = pl.ds(pl.multiple_of(sid * rpt + r * CH, 8), CH)
        pltpu.sync_copy(out_sh.at[sl], rows0)
        pltpu.sync_copy(rows0, out_hbm.at[pl.ds(
            pl.multiple_of(cid * NPAD + sid * rpt + r * CH, 8), CH)])


def _make_agg(nch):
    return pl.kernel(
        functools.partial(_agg_body, nch),
        out_type=jax.ShapeDtypeStruct((NC * NPAD, 128), _f32),
        mesh=_mesh,
        scratch_types=[
            pltpu.VMEM((CH,), jnp.int32),
            pltpu.VMEM((CH,), jnp.int32),
            pltpu.VMEM((nch, CH), jnp.int32),
            pltpu.VMEM((CH, 128), _f32),
            pltpu.VMEM((CH, 128), _f32),
            pltpu.VMEM_SHARED((NPAD, 128), _f32),
            pltpu.SemaphoreType.DMA,
            pltpu.SemaphoreType.DMA,
            pltpu.SemaphoreType.DMA,
            pltpu.SemaphoreType.DMA,
        ],
        compiler_params=pltpu.CompilerParams(needs_layout_passes=False),
    )


BR = 512
GB = NPAD // BR


def _tca_body(d_ref, h_ref, db_ref, g_ref):
    degs = jnp.sum(d_ref[...], axis=0) + 1.0
    dis = lax.rsqrt(degs)
    ones = jnp.ones((1, 128), _f32)
    parts = []
    for r in range(8):
        row = dis[r:r + 1, :]
        parts.append(lax.dot_general(
            row, ones, (((0,), (0,)), ((), ())),
            preferred_element_type=_f32))
    db = jnp.concatenate(parts, axis=0)
    db_ref[...] = db
    g_ref[...] = h_ref[...] * db


def _tch_body(x_ref, w_ref, h_ref):
    h_ref[...] = jnp.dot(x_ref[...], w_ref[...],
                         preferred_element_type=_f32)


def _tc_mid_body(e0_ref, e1_ref, g_ref, db_ref, b_ref, w_ref, go_ref):
    t = ((e0_ref[...] + e1_ref[...] + g_ref[...]) * db_ref[...]
         + b_ref[...])
    xn = jnp.maximum(t, 0.0)
    h = jnp.dot(xn, w_ref[...], preferred_element_type=_f32)
    go_ref[...] = h * db_ref[...]


def _tc_fin_body(e0_ref, e1_ref, g_ref, db_ref, b_ref, w_ref, b4_ref,
                 o_ref):
    t = ((e0_ref[...] + e1_ref[...] + g_ref[...]) * db_ref[...]
         + b_ref[...])
    xn = jnp.maximum(t, 0.0)
    o_ref[...] = jnp.dot(xn, w_ref[...],
                         preferred_element_type=_f32) + b4_ref[...]


_blk_x = pl.BlockSpec((BR, 128), lambda b: (b, 0))
_blk_w = pl.BlockSpec((128, 128), lambda b: (0, 0))
_blk_elo = pl.BlockSpec((BR, 128), lambda b: (b, 0))
_blk_ehi = pl.BlockSpec((BR, 128), lambda b: (b + GB, 0))
_blk_bias = pl.BlockSpec((1, 128), lambda b: (0, 0))


def _tca(deg3, h1):
    return pl.pallas_call(
        _tca_body,
        grid=(HR // 8,),
        in_specs=[pl.BlockSpec((NW, 8, 128), lambda b: (0, b, 0)),
                  pl.BlockSpec((1024, 128), lambda b: (b, 0))],
        out_specs=[pl.BlockSpec((1024, 128), lambda b: (b, 0)),
                   pl.BlockSpec((1024, 128), lambda b: (b, 0))],
        out_shape=[
            jax.ShapeDtypeStruct((NPAD, 128), _f32),
            jax.ShapeDtypeStruct((NPAD, 128), _f32),
        ],
    )(deg3, h1)


def _tch(x_pad, W1):
    return pl.pallas_call(
        _tch_body,
        grid=(GB,),
        in_specs=[_blk_x, _blk_w],
        out_specs=_blk_x,
        out_shape=jax.ShapeDtypeStruct((NPAD, 128), _f32),
    )(x_pad, W1)


def _tc_mid(e2, g_prev, disb, b_row, W):
    return pl.pallas_call(
        _tc_mid_body,
        grid=(GB,),
        in_specs=[_blk_elo, _blk_ehi, _blk_x, _blk_x, _blk_bias, _blk_w],
        out_specs=_blk_x,
        out_shape=jax.ShapeDtypeStruct((NPAD, 128), _f32),
    )(e2, e2, g_prev, disb, b_row, W)


def _tc_fin(e2, g_prev, disb, b_row, W4, b4_row):
    dout = W4.shape[1]
    return pl.pallas_call(
        _tc_fin_body,
        grid=(GB,),
        in_specs=[_blk_elo, _blk_ehi, _blk_x, _blk_x, _blk_bias,
                  pl.BlockSpec((128, dout), lambda b: (0, 0)),
                  pl.BlockSpec((1, dout), lambda b: (0, 0))],
        out_specs=pl.BlockSpec((BR, dout), lambda b: (b, 0)),
        out_shape=jax.ShapeDtypeStruct((NPAD, dout), _f32),
    )(e2, e2, g_prev, disb, b_row, W4, b4_row)


def kernel(x, edge_index, W1, b1, W2, b2, W3, b3, W4, b4):
    src = edge_index[0]
    dst = edge_index[1]

    nch = ((E + NW * CH - 1) // (NW * CH) + 7) // 8 * 8
    e_pad = NW * nch * CH
    padn = e_pad - E
    pidx = jnp.arange(padn, dtype=jnp.int32) % (NPAD - N)
    pad_src = N + pidx
    pad_dst = N + pidx
    src_flat = jnp.concatenate([src, pad_src])
    dst_flat = jnp.concatenate([dst, pad_dst])
    src3 = src_flat.reshape(NW, nch, CH)
    dst3 = dst_flat.reshape(NW, nch, CH)

    x_pad = jnp.concatenate(
        [x, jnp.zeros((NPAD - N, x.shape[1]), _f32)])

    h1 = _tch(x_pad, W1)
    deg3 = _make_deg(nch)(dst3).reshape(NW, HR, 128)
    disb, g1 = _tca(deg3, h1)

    b1r = b1.reshape(1, -1)
    b2r = b2.reshape(1, -1)
    b3r = b3.reshape(1, -1)
    b4r = b4.reshape(1, -1)

    agg = _make_agg(nch)
    e1 = agg(g1, src3, dst3)
    g2 = _tc_mid(e1, g1, disb, b1r, W2)
    e2 = agg(g2, src3, dst3)
    g3 = _tc_mid(e2, g2, disb, b2r, W3)
    e3 = agg(g3, src3, dst3)
    out = _tc_fin(e3, g3, disb, b3r, W4, b4r)
    return out[:N]


# speedup vs baseline: 3.8908x; 1.0103x over previous
import functools

import jax
import jax.numpy as jnp
from jax import lax
from jax.experimental import pallas as pl
from jax.experimental.pallas import tpu as pltpu
from jax.experimental.pallas import tpu_sc as plsc

NC = 2
NS = 16
NW = NC * NS
CH = 128
N = 10000
NPAD = 10240
HR = NPAD // 128
E = 320000

_mesh = plsc.VectorSubcoreMesh(core_axis_name="c", subcore_axis_name="s")
_f32 = jnp.float32


def _zero16():
    return jnp.zeros((16,), _f32)


def _ones16():
    return jnp.ones((16,), _f32)


def _deg_body(nch, dst_hbm, out_hbm, db0, db1, hist, sd0, sd1):
    cid = lax.axis_index("c")
    sid = lax.axis_index("s")
    wid = sid * NC + cid

    def _zrow(i, c):
        hist[pl.ds(pl.multiple_of(i * 16, 8), 16)] = _zero16()
        return c

    lax.fori_loop(0, NPAD // 16, _zrow, 0)

    dbs = (db0, db1)
    sds = (sd0, sd1)
    pltpu.async_copy(dst_hbm.at[wid, 0], db0, sd0)
    pltpu.async_copy(dst_hbm.at[wid, 1], db1, sd1)

    def _pair(i, carry):
        for k in range(2):
            c = i * 2 + k
            pltpu.make_async_copy(dst_hbm.at[wid, c], dbs[k],
                                  sds[k]).wait()
            for j in range(CH // 16):
                dv = dbs[k][pl.ds(j * 16, 16)]
                plsc.addupdate_scatter(hist, [dv], _ones16())
            nxt = jnp.minimum(c + 2, nch - 1)
            pltpu.async_copy(dst_hbm.at[wid, nxt], dbs[k], sds[k])
        return carry

    lax.fori_loop(0, nch // 2, _pair, 0)
    pltpu.make_async_copy(dst_hbm.at[wid, 0], db0, sd0).wait()
    pltpu.make_async_copy(dst_hbm.at[wid, 0], db1, sd1).wait()

    pltpu.sync_copy(hist,
                    out_hbm.at[pl.ds(pl.multiple_of(wid * NPAD, 8), NPAD)])


def _make_deg(nch):
    return pl.kernel(
        functools.partial(_deg_body, nch),
        out_type=jax.ShapeDtypeStruct((NW * NPAD,), _f32),
        mesh=_mesh,
        scratch_types=[
            pltpu.VMEM((CH,), jnp.int32),
            pltpu.VMEM((CH,), jnp.int32),
            pltpu.VMEM((NPAD,), _f32),
            pltpu.SemaphoreType.DMA,
            pltpu.SemaphoreType.DMA,
        ],
        compiler_params=pltpu.CompilerParams(needs_layout_passes=False),
    )


def _agg_body(nch, g_hbm, src_hbm, dst_hbm, out_hbm, sb0, sb1, dstv,
              rows0, rows1, out_sh, gs0, gs1, ss0, ss1):
    cid = lax.axis_index("c")
    sid = lax.axis_index("s")
    wid = sid * NC + cid
    rpt = NPAD // NS

    pltpu.sync_copy(dst_hbm.at[wid], dstv)

    def _zrow(i, c):
        for j in range(8):
            rows0[i, pl.ds(j * 16, 16)] = _zero16()
        return c

    lax.fori_loop(0, CH, _zrow, 0)
    for r in range(rpt // CH):
        pltpu.async_copy(rows0, out_sh.at[pl.ds(
            pl.multiple_of(sid * rpt + r * CH, 8), CH)], gs1)
    for r in range(rpt // CH):
        pltpu.make_async_copy(rows0, out_sh.at[pl.ds(
            pl.multiple_of(sid * rpt, 8), CH)], gs1).wait()
    plsc.subcore_barrier()

    sbs = (sb0, sb1)
    rbs = (rows0, rows1)
    gss = (gs0, gs1)
    sss = (ss0, ss1)

    pltpu.sync_copy(src_hbm.at[wid, 0], sb0)
    pltpu.async_copy(g_hbm.at[sb0], rows0, gs0)
    pltpu.async_copy(src_hbm.at[wid, 1], sb1, ss1)

    def _pair(i, carry):
        for k in range(2):
            c = i * 2 + k
            ko = (k + 1) % 2
            pltpu.make_async_copy(src_hbm.at[wid, c], sbs[ko],
                                  sss[ko]).wait()
            pltpu.async_copy(g_hbm.at[sbs[ko]], rbs[ko], gss[ko])
            pltpu.make_async_copy(g_hbm.at[sbs[k]], rbs[k], gss[k]).wait()
            pltpu.sync_copy(rbs[k], out_sh.at[dstv.at[c]], add=True)
            nxt = jnp.minimum(c + 2, nch - 1)
            pltpu.async_copy(src_hbm.at[wid, nxt], sbs[k], sss[k])
        return carry

    lax.fori_loop(0, nch // 2, _pair, 0)
    pltpu.make_async_copy(g_hbm.at[sbs[0]], rbs[0], gss[0]).wait()
    pltpu.make_async_copy(src_hbm.at[wid, 0], sbs[1], sss[1]).wait()
    plsc.subcore_barrier()

    nwb = rpt // CH
    for r in range(nwb):
        k = r % 2
        hb = pl.ds(pl.multiple_of(cid * NPAD + sid * rpt + r * CH, 8), CH)
        if r >= 2:
            pltpu.make_async_copy(rbs[k], out_hbm.at[hb], sss[k]).wait()
        pltpu.sync_copy(out_sh.at[pl.ds(
            pl.multiple_of(sid * rpt + r * CH, 8), CH)], rbs[k])
        pltpu.async_copy(rbs[k], out_hbm.at[hb], sss[k])
    for r in range(nwb - 2, nwb):
        k = r % 2
        hb = pl.ds(pl.multiple_of(cid * NPAD + sid * rpt + r * CH, 8), CH)
        pltpu.make_async_copy(rbs[k], out_hbm.at[hb], sss[k]).wait()


def _make_agg(nch):
    return pl.kernel(
        functools.partial(_agg_body, nch),
        out_type=jax.ShapeDtypeStruct((NC * NPAD, 128), _f32),
        mesh=_mesh,
        scratch_types=[
            pltpu.VMEM((CH,), jnp.int32),
            pltpu.VMEM((CH,), jnp.int32),
            pltpu.VMEM((nch, CH), jnp.int32),
            pltpu.VMEM((CH, 128), _f32),
            pltpu.VMEM((CH, 128), _f32),
            pltpu.VMEM_SHARED((NPAD, 128), _f32),
            pltpu.SemaphoreType.DMA,
            pltpu.SemaphoreType.DMA,
            pltpu.SemaphoreType.DMA,
            pltpu.SemaphoreType.DMA,
        ],
        compiler_params=pltpu.CompilerParams(needs_layout_passes=False),
    )


BR = 512
GB = NPAD // BR


def _tca_body(d_ref, h_ref, db_ref, g_ref):
    degs = jnp.sum(d_ref[...], axis=0) + 1.0
    dis = lax.rsqrt(degs)
    ones = jnp.ones((1, 128), _f32)
    parts = []
    for r in range(8):
        row = dis[r:r + 1, :]
        parts.append(lax.dot_general(
            row, ones, (((0,), (0,)), ((), ())),
            preferred_element_type=_f32))
    db = jnp.concatenate(parts, axis=0)
    db_ref[...] = db
    g_ref[...] = h_ref[...] * db


def _tch_body(x_ref, w_ref, h_ref):
    h_ref[...] = jnp.dot(x_ref[...], w_ref[...],
                         preferred_element_type=_f32)


def _tc_mid_body(e0_ref, e1_ref, g_ref, db_ref, b_ref, w_ref, go_ref):
    t = ((e0_ref[...] + e1_ref[...] + g_ref[...]) * db_ref[...]
         + b_ref[...])
    xn = jnp.maximum(t, 0.0)
    h = jnp.dot(xn, w_ref[...], preferred_element_type=_f32)
    go_ref[...] = h * db_ref[...]


def _tc_fin_body(e0_ref, e1_ref, g_ref, db_ref, b_ref, w_ref, b4_ref,
                 o_ref):
    t = ((e0_ref[...] + e1_ref[...] + g_ref[...]) * db_ref[...]
         + b_ref[...])
    xn = jnp.maximum(t, 0.0)
    o_ref[...] = jnp.dot(xn, w_ref[...],
                         preferred_element_type=_f32) + b4_ref[...]


_blk_x = pl.BlockSpec((BR, 128), lambda b: (b, 0))
_blk_w = pl.BlockSpec((128, 128), lambda b: (0, 0))
_blk_elo = pl.BlockSpec((BR, 128), lambda b: (b, 0))
_blk_ehi = pl.BlockSpec((BR, 128), lambda b: (b + GB, 0))
_blk_bias = pl.BlockSpec((1, 128), lambda b: (0, 0))


def _tca(deg3, h1):
    return pl.pallas_call(
        _tca_body,
        grid=(HR // 8,),
        in_specs=[pl.BlockSpec((NW, 8, 128), lambda b: (0, b, 0)),
                  pl.BlockSpec((1024, 128), lambda b: (b, 0))],
        out_specs=[pl.BlockSpec((1024, 128), lambda b: (b, 0)),
                   pl.BlockSpec((1024, 128), lambda b: (b, 0))],
        out_shape=[
            jax.ShapeDtypeStruct((NPAD, 128), _f32),
            jax.ShapeDtypeStruct((NPAD, 128), _f32),
        ],
    )(deg3, h1)


def _tch(x_pad, W1):
    return pl.pallas_call(
        _tch_body,
        grid=(GB,),
        in_specs=[_blk_x, _blk_w],
        out_specs=_blk_x,
        out_shape=jax.ShapeDtypeStruct((NPAD, 128), _f32),
    )(x_pad, W1)


def _tc_mid(e2, g_prev, disb, b_row, W):
    return pl.pallas_call(
        _tc_mid_body,
        grid=(GB,),
        in_specs=[_blk_elo, _blk_ehi, _blk_x, _blk_x, _blk_bias, _blk_w],
        out_specs=_blk_x,
        out_shape=jax.ShapeDtypeStruct((NPAD, 128), _f32),
    )(e2, e2, g_prev, disb, b_row, W)


def _tc_fin(e2, g_prev, disb, b_row, W4, b4_row):
    dout = W4.shape[1]
    return pl.pallas_call(
        _tc_fin_body,
        grid=(GB,),
        in_specs=[_blk_elo, _blk_ehi, _blk_x, _blk_x, _blk_bias,
                  pl.BlockSpec((128, dout), lambda b: (0, 0)),
                  pl.BlockSpec((1, dout), lambda b: (0, 0))],
        out_specs=pl.BlockSpec((BR, dout), lambda b: (b, 0)),
        out_shape=jax.ShapeDtypeStruct((NPAD, dout), _f32),
    )(e2, e2, g_prev, disb, b_row, W4, b4_row)


def kernel(x, edge_index, W1, b1, W2, b2, W3, b3, W4, b4):
    src = edge_index[0]
    dst = edge_index[1]

    nch = ((E + NW * CH - 1) // (NW * CH) + 7) // 8 * 8
    e_pad = NW * nch * CH
    padn = e_pad - E
    pidx = jnp.arange(padn, dtype=jnp.int32) % (NPAD - N)
    pad_src = N + pidx
    pad_dst = N + pidx
    src_flat = jnp.concatenate([src, pad_src])
    dst_flat = jnp.concatenate([dst, pad_dst])
    src3 = src_flat.reshape(NW, nch, CH)
    dst3 = dst_flat.reshape(NW, nch, CH)

    x_pad = jnp.concatenate(
        [x, jnp.zeros((NPAD - N, x.shape[1]), _f32)])

    h1 = _tch(x_pad, W1)
    deg3 = _make_deg(nch)(dst3).reshape(NW, HR, 128)
    disb, g1 = _tca(deg3, h1)

    b1r = b1.reshape(1, -1)
    b2r = b2.reshape(1, -1)
    b3r = b3.reshape(1, -1)
    b4r = b4.reshape(1, -1)

    agg = _make_agg(nch)
    e1 = agg(g1, src3, dst3)
    g2 = _tc_mid(e1, g1, disb, b1r, W2)
    e2 = agg(g2, src3, dst3)
    g3 = _tc_mid(e2, g2, disb, b2r, W3)
    e3 = agg(g3, src3, dst3)
    out = _tc_fin(e3, g3, disb, b3r, W4, b4r)
    return out[:N]
